# baseline (speedup 1.0000x reference)
import functools

import jax
import jax.numpy as jnp
from jax import lax
from jax.experimental import pallas as pl
from jax.experimental.pallas import tpu as pltpu
from jax.experimental.pallas import tpu_sc as plsc

_BATCH = 16384
_FIELDS = 26
_DIM = 64
_NUM_WORKERS = 32
_ROWS_PER_WORKER = _BATCH // _NUM_WORKERS
_SUB = 16
_NUM_SUB = _ROWS_PER_WORKER // _SUB
_NBUF = 2
_NOUTER = _NUM_SUB // _NBUF

_mesh = plsc.VectorSubcoreMesh(core_axis_name="c", subcore_axis_name="s")


@functools.partial(
    pl.kernel,
    out_type=jax.ShapeDtypeStruct((_FIELDS, 8, 128, 8, 128), jnp.float32),
    mesh=_mesh,
    scratch_types=[
        pltpu.VMEM((_ROWS_PER_WORKER, _FIELDS), jnp.int32),
        pltpu.VMEM((_NBUF, _SUB, _FIELDS, _DIM), jnp.float32),
        pltpu.VMEM((_NBUF, _FIELDS, 8, 8, _SUB), jnp.float32),
        pltpu.SemaphoreType.DMA,
        pltpu.SemaphoreType.DMA,
        pltpu.SemaphoreType.DMA,
        pltpu.SemaphoreType.DMA,
    ],
    compiler_params=pltpu.CompilerParams(
        use_tc_tiling_on_sc=False, needs_layout_passes=False),
)
def _gather_kernel(keys_hbm, table_hbm, out_hbm, idx_v, rows_v, pbuf_v,
                   gs0, gs1, os0, os1):
    gsem = (gs0, gs1)
    osem = (os0, os1)
    wid = lax.axis_index("s") * 2 + lax.axis_index("c")
    base = wid * _ROWS_PER_WORKER
    pltpu.sync_copy(keys_hbm.at[pl.ds(base, _ROWS_PER_WORKER)], idx_v)

    def gather(j, b):
        def start():
            for i in range(_SUB):
                pltpu.make_async_copy(
                    table_hbm.at[idx_v.at[j * _SUB + i]],
                    rows_v.at[b].at[i], gsem[b]).start()

        def wait():
            for i in range(_SUB):
                pltpu.make_async_copy(
                    table_hbm.at[idx_v.at[j * _SUB + i]],
                    rows_v.at[b].at[i], gsem[b]).wait()

        return start, wait

    def transpose(b):
        lanes = lax.iota(jnp.int32, 16)
        zeros = jnp.zeros((16,), jnp.int32)

        def per_field(f, carry):
            f_vec = zeros + f
            for t in range(8):
                for s in range(8):
                    v = plsc.load_gather(
                        rows_v.at[b], [lanes, f_vec, zeros + (8 * t + s)])
                    plsc.store_scatter(
                        pbuf_v.at[b], [f_vec, zeros + t, zeros + s, lanes], v)
            return carry

        lax.fori_loop(0, _FIELDS, per_field, 0)

    def store(j, b):
        c = (base + j * _SUB) // 128
        l0 = (j * _SUB) % 128
        return pltpu.make_async_copy(
            pbuf_v.at[b],
            out_hbm.at[:, :, c, :, pl.ds(l0, _SUB)], osem[b])

    for b in range(_NBUF):
        gather(b, b)[0]()

    def body(i, carry):
        for b in range(_NBUF):
            j = i * _NBUF + b
            gather(j, b)[1]()
            transpose(b)
            store(j, b).start()
        for b in range(_NBUF):
            j = i * _NBUF + b
            store(j, b).wait()
            gather(j + _NBUF, b)[0]()
        return carry

    lax.fori_loop(0, _NOUTER - 1, body, 0)

    for b in range(_NBUF):
        j = (_NOUTER - 1) * _NBUF + b
        gather(j, b)[1]()
        transpose(b)
        store(j, b).start()
    for b in range(_NBUF):
        j = (_NOUTER - 1) * _NBUF + b
        store(j, b).wait()


def kernel(keys, table):
    out5 = _gather_kernel(keys, table)
    return out5.transpose((2, 4, 0, 1, 3)).reshape(_BATCH, _FIELDS, _DIM)
